# baseline (speedup 1.0000x reference)
import functools

import jax
import jax.numpy as jnp
from jax import lax
from jax.experimental import pallas as pl
from jax.experimental.pallas import tpu as pltpu
from jax.experimental.pallas import tpu_sc as plsc

_INFO = plsc.get_sparse_core_info()
_NC = _INFO.num_cores
_NS = _INFO.num_subcores
_NW = _NC * _NS

_CH = 128
_NBUF = 4


def _gather_call(n_chunks, D, B):
    mesh = plsc.VectorSubcoreMesh(core_axis_name="c", subcore_axis_name="s")

    @functools.partial(
        pl.kernel,
        mesh=mesh,
        out_type=jax.ShapeDtypeStruct((B, D), jnp.float32),
        compiler_params=pltpu.CompilerParams(use_tc_tiling_on_sc=False),
        scratch_types=[
            pltpu.VMEM((n_chunks, _CH), jnp.int32),
            pltpu.VMEM((_NBUF, _CH, D), jnp.float32),
            pltpu.SemaphoreType.DMA((_NBUF,)),
        ],
    )
    def body(idx_hbm, table_hbm, out_hbm, idx_v, bufs, sems):
        wid = lax.axis_index("s") * _NC + lax.axis_index("c")
        pltpu.sync_copy(idx_hbm.at[wid], idx_v)

        base = wid * (n_chunks * _CH)

        def issue(j, b):
            pltpu.async_copy(table_hbm.at[idx_v.at[j]], bufs.at[b], sems.at[b])

        def wait(b):
            pltpu.make_async_copy(
                table_hbm.at[pl.ds(0, _CH)], bufs.at[b], sems.at[b]
            ).wait()

        for b in range(_NBUF):
            issue(b, b)

        def group(g, carry):
            for b in range(_NBUF):
                j = g * _NBUF + b
                wait(b)
                pltpu.sync_copy(bufs.at[b], out_hbm.at[pl.ds(base + j * _CH, _CH)])
                issue((j + _NBUF) % n_chunks, b)
            return carry

        lax.fori_loop(0, n_chunks // _NBUF, group, 0)
        for b in range(_NBUF):
            wait(b)

    return body


def kernel(inlets, weight):
    b0, b1 = inlets.shape
    V, D = weight.shape
    B = b0 * b1
    per_w = B // _NW
    n_chunks = per_w // _CH
    idx = inlets.reshape(_NW, n_chunks, _CH).astype(jnp.int32)
    out = _gather_call(n_chunks, D, B)(idx, weight)
    return out.reshape(b0, b1, D)
